# Initial kernel scaffold; baseline (speedup 1.0000x reference)
import jax
import jax.numpy as jnp
from jax.experimental import pallas as pl


def kernel(x, edge_index, edge_attr, params):
    raise NotImplementedError("write your pallas kernel here")


# speedup vs baseline: 1.0358x; 1.0358x over previous
import functools

import jax
import jax.numpy as jnp
from jax.experimental import pallas as pl

N = 50000
E = 800000
D = 64
HID = 64
HEADS = 4
DH = 16


def _dense_in_kernel(x_ref, w_ref, b_ref, o_ref):
    o_ref[...] = jax.nn.relu(x_ref[...] @ w_ref[...] + b_ref[...])


def _dense_in(x, w, b):
    BN = 2000
    return pl.pallas_call(
        _dense_in_kernel,
        grid=(N // BN,),
        in_specs=[
            pl.BlockSpec((BN, D), lambda i: (i, 0)),
            pl.BlockSpec((D, HID), lambda i: (0, 0)),
            pl.BlockSpec((1, HID), lambda i: (0, 0)),
        ],
        out_specs=pl.BlockSpec((BN, HID), lambda i: (i, 0)),
        out_shape=jax.ShapeDtypeStruct((N, HID), jnp.float32),
    )(x, w, b.reshape(1, HID))


def kernel(x, edge_index, edge_attr, params):
    src = edge_index[0]
    dst = edge_index[1]
    scale = DH ** -0.5
    h = _dense_in(x, params['W_in'], params['b_in'])
    for i in range(3):
        p = params['layers'][i]
        n = h.shape[0]
        Q = (h @ p['Wq']).reshape(n, HEADS, DH)
        K = (h @ p['Wk']).reshape(n, HEADS, DH)
        V = (h @ p['Wv']).reshape(n, HEADS, DH)
        edge_bias = edge_attr @ p['We']
        attn = (Q[dst] * K[src]).sum(axis=-1) * scale + edge_bias
        attn = jnp.where(attn >= 0, attn, 0.2 * attn)
        attn_exp = jnp.exp(attn)
        attn_sum = jnp.zeros((n, HEADS), jnp.float32).at[dst].add(attn_exp)
        weighted_V = V[src] * attn_exp[:, :, None]
        agg = jnp.zeros((n, HEADS, DH), jnp.float32).at[dst].add(weighted_V)
        agg = agg / jnp.maximum(attn_sum, 1e-12)[:, :, None]
        agg = agg.reshape(n, HEADS * DH) @ p['Wo'] + p['bo']
        cat = jnp.concatenate([h, agg], axis=1)
        hc = jax.nn.relu(cat @ p['Wm'] + p['bm'])
        hr = h + hc
        m = hr.mean(axis=-1, keepdims=True)
        v = ((hr - m) ** 2).mean(axis=-1, keepdims=True)
        h = p['g'] * (hr - m) / jnp.sqrt(v + 1e-5) + p['b']
    h1 = jax.nn.relu(h @ params['Wh1'] + params['bh1'])
    return h1 @ params['Wh2'] + params['bh2']


# speedup vs baseline: 119.6733x; 115.5368x over previous
import functools

import jax
import jax.numpy as jnp
import numpy as np
from jax import lax
from jax.experimental import pallas as pl
from jax.experimental.pallas import tpu as pltpu
from jax.experimental.pallas import tpu_sc as plsc

N = 50000
E = 800000
D = 64
HID = 64
HEADS = 4
DH = 16
SCALE = DH ** -0.5

NCORE = 2
NSUB = 16

GB = 128
SB = 40
PW = 40
NPT = N // NSUB
HP = jax.lax.Precision.HIGHEST

_SEL = np.kron(np.eye(HEADS, dtype=np.float32), np.ones((DH, 1), np.float32))


def _dense_in_body(x_ref, w_ref, b_ref, o_ref):
    o_ref[...] = jnp.maximum(x_ref[...] @ w_ref[...] + b_ref[...], 0.0)


def _dense_in(x, w, b):
    BN = 2000
    return pl.pallas_call(
        _dense_in_body,
        grid=(N // BN,),
        in_specs=[
            pl.BlockSpec((BN, D), lambda i: (i, 0)),
            pl.BlockSpec((D, HID), lambda i: (0, 0)),
            pl.BlockSpec((1, HID), lambda i: (0, 0)),
        ],
        out_specs=pl.BlockSpec((BN, HID), lambda i: (i, 0)),
        out_shape=jax.ShapeDtypeStruct((N, HID), jnp.float32),
    )(x, w, b.reshape(1, HID))


def _qkv_body(h_ref, wq_ref, wkv_ref, q_ref, kv_ref):
    h = h_ref[...]
    q_ref[...] = h @ wq_ref[...]
    kv_ref[...] = h @ wkv_ref[...]


def _qkv(h, wq, wkv):
    BN = 2000
    return pl.pallas_call(
        _qkv_body,
        grid=(N // BN,),
        in_specs=[
            pl.BlockSpec((BN, HID), lambda i: (i, 0)),
            pl.BlockSpec((HID, 64), lambda i: (0, 0)),
            pl.BlockSpec((HID, 128), lambda i: (0, 0)),
        ],
        out_specs=[
            pl.BlockSpec((BN, 64), lambda i: (i, 0)),
            pl.BlockSpec((BN, 128), lambda i: (i, 0)),
        ],
        out_shape=[
            jax.ShapeDtypeStruct((N, 64), jnp.float32),
            jax.ShapeDtypeStruct((N, 128), jnp.float32),
        ],
    )(h, wq, wkv)


def _edge_body(qd_ref, kvs_ref, ea_ref, we_ref, sel_ref, selt_ref, o_ref):
    qd = qd_ref[...]
    kv = kvs_ref[...]
    ks = kv[:, :64]
    vs = kv[:, 64:]
    m = qd * ks
    s = (jnp.dot(m, sel_ref[...], precision=HP) * SCALE
         + jnp.dot(ea_ref[...], we_ref[...], precision=HP))
    s = jnp.where(s >= 0.0, s, 0.2 * s)
    p = jnp.exp(s)
    w = vs * jnp.dot(p, selt_ref[...], precision=HP)
    z = jnp.zeros((w.shape[0], 4), jnp.float32)
    o_ref[0] = jnp.concatenate([w[:, :32], p, z], axis=1)
    o_ref[1] = jnp.concatenate([w[:, 32:], p, z], axis=1)


def _edge_math(qd, kvs, edge_attr, we):
    BE = 2000
    sel = jnp.asarray(_SEL)
    return pl.pallas_call(
        _edge_body,
        grid=(E // BE,),
        in_specs=[
            pl.BlockSpec((BE, 64), lambda i: (i, 0)),
            pl.BlockSpec((BE, 128), lambda i: (i, 0)),
            pl.BlockSpec((BE, 3), lambda i: (i, 0)),
            pl.BlockSpec((3, HEADS), lambda i: (0, 0)),
            pl.BlockSpec((64, HEADS), lambda i: (0, 0)),
            pl.BlockSpec((HEADS, 64), lambda i: (0, 0)),
        ],
        out_specs=pl.BlockSpec((2, BE, PW), lambda i: (0, i, 0)),
        out_shape=jax.ShapeDtypeStruct((2, E, PW), jnp.float32),
    )(qd, kvs, edge_attr, we, sel, sel.T)


def _post_body(h_ref, wp_ref, wo_ref, bo_ref, wmh_ref, wma_ref, bm_ref,
               g_ref, b_ref, selt_ref, o_ref):
    wp0 = wp_ref[0]
    wp1 = wp_ref[1]
    ps = jnp.concatenate([wp0[:, 32:34], wp1[:, 34:36]], axis=1)
    den = jnp.maximum(ps, 1e-12)
    agg = jnp.concatenate([wp0[:, :32], wp1[:, :32]], axis=1)
    agg = agg / jnp.dot(den, selt_ref[...], precision=HP)
    agg2 = agg @ wo_ref[...] + bo_ref[...]
    h = h_ref[...]
    hc = jnp.maximum(h @ wmh_ref[...] + agg2 @ wma_ref[...] + bm_ref[...], 0.0)
    hr = h + hc
    mu = jnp.mean(hr, axis=1, keepdims=True)
    var = jnp.mean((hr - mu) ** 2, axis=1, keepdims=True)
    o_ref[...] = g_ref[...] * (hr - mu) * lax.rsqrt(var + 1e-5) + b_ref[...]


def _post(h, wp, p):
    BN = 2000
    selt = jnp.asarray(_SEL.T)
    row = lambda a: a.reshape(1, HID)
    return pl.pallas_call(
        _post_body,
        grid=(N // BN,),
        in_specs=[
            pl.BlockSpec((BN, HID), lambda i: (i, 0)),
            pl.BlockSpec((2, BN, PW), lambda i: (0, i, 0)),
            pl.BlockSpec((64, HID), lambda i: (0, 0)),
            pl.BlockSpec((1, HID), lambda i: (0, 0)),
            pl.BlockSpec((HID, HID), lambda i: (0, 0)),
            pl.BlockSpec((HID, HID), lambda i: (0, 0)),
            pl.BlockSpec((1, HID), lambda i: (0, 0)),
            pl.BlockSpec((1, HID), lambda i: (0, 0)),
            pl.BlockSpec((1, HID), lambda i: (0, 0)),
            pl.BlockSpec((HEADS, 64), lambda i: (0, 0)),
        ],
        out_specs=pl.BlockSpec((BN, HID), lambda i: (i, 0)),
        out_shape=jax.ShapeDtypeStruct((N, HID), jnp.float32),
    )(h, wp, p['Wo'], row(p['bo']), p['Wm'][:HID], p['Wm'][HID:],
      row(p['bm']), row(p['g']), row(p['b']), selt)


def _head_body(h_ref, w1_ref, b1_ref, w2_ref, b2_ref, o_ref):
    h1 = jnp.maximum(h_ref[...] @ w1_ref[...] + b1_ref[...], 0.0)
    o_ref[...] = h1 @ w2_ref[...] + b2_ref[...]


def _head(h, w1, b1, w2, b2):
    BN = 2000
    return pl.pallas_call(
        _head_body,
        grid=(N // BN,),
        in_specs=[
            pl.BlockSpec((BN, HID), lambda i: (i, 0)),
            pl.BlockSpec((HID, HID // 2), lambda i: (0, 0)),
            pl.BlockSpec((1, HID // 2), lambda i: (0, 0)),
            pl.BlockSpec((HID // 2, 1), lambda i: (0, 0)),
            pl.BlockSpec((1, 1), lambda i: (0, 0)),
        ],
        out_specs=pl.BlockSpec((BN, 1), lambda i: (i, 0)),
        out_shape=jax.ShapeDtypeStruct((N, 1), jnp.float32),
    )(h, w1, b1.reshape(1, HID // 2), w2, b2.reshape(1, 1))


_MESH = plsc.VectorSubcoreMesh(core_axis_name="c", subcore_axis_name="s")
_SC_PARAMS = pltpu.CompilerParams(use_tc_tiling_on_sc=False)


@jax.jit
def _sc_gather(q, kv, dst1, src1):
    @functools.partial(
        pl.kernel,
        mesh=_MESH,
        compiler_params=_SC_PARAMS,
        out_type=[
            jax.ShapeDtypeStruct((E, 64), jnp.float32),
            jax.ShapeDtypeStruct((E, 128), jnp.float32),
        ],
    )
    def k(q_hbm, kv_hbm, dst_hbm, src_hbm, qd_out, kvs_out):
        def body(dst_v, src_v, qd_v, kvs_v):
            pltpu.sync_copy(q_hbm.at[dst_v.at[0]], qd_v)
            pltpu.sync_copy(kv_hbm.at[src_v.at[0]], kvs_v)

        pltpu.emit_pipeline(
            body,
            grid=(E // GB,),
            in_specs=[
                pl.BlockSpec((1, GB), lambda i: (0, i)),
                pl.BlockSpec((1, GB), lambda i: (0, i)),
            ],
            out_specs=[
                pl.BlockSpec((GB, 64), lambda i: (i, 0)),
                pl.BlockSpec((GB, 128), lambda i: (i, 0)),
            ],
            core_axis_name=("c", "s"),
            dimension_semantics=(pltpu.PARALLEL,),
        )(dst_hbm, src_hbm, qd_out, kvs_out)

    return k(q, kv, dst1, src1)


@jax.jit
def _sc_scatter(wp, dst2, zeros):
    @functools.partial(
        pl.kernel,
        mesh=_MESH,
        compiler_params=_SC_PARAMS,
        out_type=jax.ShapeDtypeStruct((2, N, PW), jnp.float32),
        scratch_types=[
            pltpu.VMEM_SHARED((N, PW), jnp.float32),
        ],
    )
    def k(wp_hbm, dst_hbm, z_hbm, out_hbm, acc):
        c = lax.axis_index("c")
        s = lax.axis_index("s")
        pltpu.sync_copy(z_hbm.at[pl.ds(s * NPT, NPT)],
                        acc.at[pl.ds(s * NPT, NPT)])
        plsc.subcore_barrier()

        def body(ib_v, wb_v):
            pltpu.sync_copy(wb_v, acc.at[ib_v.at[0]], add=True)

        pltpu.emit_pipeline(
            body,
            grid=(E // SB,),
            in_specs=[
                pl.BlockSpec((1, SB), lambda i: (0, i)),
                pl.BlockSpec((SB, PW), lambda i: (i, 0)),
            ],
            core_axis_name="s",
            dimension_semantics=(pltpu.PARALLEL,),
        )(dst_hbm, wp_hbm.at[c])

        plsc.subcore_barrier()
        pltpu.sync_copy(acc.at[pl.ds(s * NPT, NPT)],
                        out_hbm.at[c].at[pl.ds(s * NPT, NPT)])

    return k(wp, dst2, zeros)


def kernel(x, edge_index, edge_attr, params):
    src1 = edge_index[0].reshape(1, E)
    dst1 = edge_index[1].reshape(1, E)
    zeros = jnp.zeros((N, PW), jnp.float32)

    h = _dense_in(x, params['W_in'], params['b_in'])
    for i in range(3):
        p = params['layers'][i]
        wkv = jnp.concatenate([p['Wk'], p['Wv']], axis=1)
        q, kv = _qkv(h, p['Wq'], wkv)
        qd, kvs = _sc_gather(q, kv, dst1, src1)
        wp = _edge_math(qd, kvs, edge_attr, p['We'])
        acc = _sc_scatter(wp, dst1, zeros)
        h = _post(h, acc, p)
    return _head(h, params['Wh1'], params['bh1'], params['Wh2'], params['bh2'])
